# Initial kernel scaffold; baseline (speedup 1.0000x reference)
import jax
import jax.numpy as jnp
from jax.experimental import pallas as pl


def kernel(in_field, scales, shifts, species_idx, ptr):
    raise NotImplementedError("write your pallas kernel here")


# speedup vs baseline: 3.5271x; 3.5271x over previous
import functools

import jax
import jax.numpy as jnp
from jax import lax
from jax.experimental import pallas as pl
from jax.experimental.pallas import tpu as pltpu
from jax.experimental.pallas import tpu_sc as plsc

L = 16


def _scale_shift_call(x, sc, sh, sp, n, chunk, n_full, rem, nc, ns):
    nw = nc * ns
    full_iters = chunk // L
    rem_iters = rem // L

    mesh = plsc.VectorSubcoreMesh(core_axis_name="c", subcore_axis_name="s")

    @functools.partial(
        pl.kernel,
        mesh=mesh,
        out_type=jax.ShapeDtypeStruct((n,), jnp.float32),
        compiler_params=pltpu.CompilerParams(needs_layout_passes=False),
        scratch_types=[
            pltpu.VMEM((chunk,), jnp.int32),
            pltpu.VMEM((chunk,), jnp.float32),
            pltpu.VMEM((chunk,), jnp.float32),
            pltpu.VMEM((64,), jnp.float32),
            pltpu.VMEM((64,), jnp.float32),
        ],
    )
    def run(x_hbm, sc_hbm, sh_hbm, sp_hbm, out_hbm, sp_v, x_v, o_v, sc_v, sh_v):
        wid = lax.axis_index("s") * nc + lax.axis_index("c")
        base = wid * chunk

        pltpu.sync_copy(sc_hbm, sc_v)
        pltpu.sync_copy(sh_hbm, sh_v)

        @pl.when(wid < n_full)
        def _():
            pltpu.sync_copy(sp_hbm.at[pl.ds(base, chunk)], sp_v)
            pltpu.sync_copy(x_hbm.at[pl.ds(base, chunk)], x_v)

        if rem > 0:

            @pl.when(wid == n_full)
            def _():
                pltpu.sync_copy(sp_hbm.at[pl.ds(base, rem)], sp_v.at[pl.ds(0, rem)])
                pltpu.sync_copy(x_hbm.at[pl.ds(base, rem)], x_v.at[pl.ds(0, rem)])

        my_iters = jnp.where(
            wid < n_full, full_iters, jnp.where(wid == n_full, rem_iters, 0)
        )

        def body(i, carry):
            sl = pl.ds(i * L, L)
            idx = sp_v[sl]
            s = plsc.load_gather(sc_v, [idx])
            t = plsc.load_gather(sh_v, [idx])
            o_v[sl] = s * x_v[sl] + t
            return carry

        lax.fori_loop(0, my_iters, body, 0)

        @pl.when(wid < n_full)
        def _():
            pltpu.sync_copy(o_v, out_hbm.at[pl.ds(base, chunk)])

        if rem > 0:

            @pl.when(wid == n_full)
            def _():
                pltpu.sync_copy(o_v.at[pl.ds(0, rem)], out_hbm.at[pl.ds(base, rem)])

    return run(x, sc, sh, sp)


def kernel(in_field, scales, shifts, species_idx, ptr):
    del ptr
    n0 = in_field.shape[0]
    x = in_field.reshape(-1)
    sp = species_idx.reshape(-1).astype(jnp.int32)
    sc = scales.reshape(-1).astype(jnp.float32)
    sh = shifts.reshape(-1).astype(jnp.float32)

    info = plsc.get_sparse_core_info()
    nc, ns = info.num_cores, info.num_subcores
    nw = nc * ns

    n = n0
    if n % L != 0:
        n = (n0 + L - 1) // L * L
        x = jnp.pad(x, (0, n - n0))
        sp = jnp.pad(sp, (0, n - n0))

    per_worker = -(-n // nw)
    chunk = -(-per_worker // L) * L
    n_full = n // chunk
    rem = n - n_full * chunk

    out = _scale_shift_call(x, sc, sh, sp, n, chunk, n_full, rem, nc, ns)
    return out[:n0].reshape(n0, 1)


# speedup vs baseline: 3.8771x; 1.0992x over previous
import functools

import jax
import jax.numpy as jnp
from jax import lax
from jax.experimental import pallas as pl
from jax.experimental.pallas import tpu as pltpu
from jax.experimental.pallas import tpu_sc as plsc

L = 16


def _scale_shift_call(x, sc, sh, sp, n, chunk, nc, ns):
    iters = chunk // L

    mesh = plsc.VectorSubcoreMesh(core_axis_name="c", subcore_axis_name="s")

    @functools.partial(
        pl.kernel,
        mesh=mesh,
        out_type=jax.ShapeDtypeStruct((n,), jnp.float32),
        compiler_params=pltpu.CompilerParams(needs_layout_passes=False),
        scratch_types=[
            pltpu.VMEM((chunk,), jnp.int32),
            pltpu.VMEM((chunk,), jnp.float32),
            pltpu.VMEM((chunk,), jnp.float32),
            pltpu.VMEM((64,), jnp.float32),
            pltpu.VMEM((64,), jnp.float32),
            pltpu.SemaphoreType.DMA,
        ],
    )
    def run(x_hbm, sc_hbm, sh_hbm, sp_hbm, out_hbm, sp_v, x_v, o_v, sc_v, sh_v, sem):
        wid = lax.axis_index("s") * nc + lax.axis_index("c")
        base = jnp.minimum(wid * chunk, n - chunk)

        a = pltpu.async_copy(sp_hbm.at[pl.ds(base, chunk)], sp_v, sem)
        b = pltpu.async_copy(x_hbm.at[pl.ds(base, chunk)], x_v, sem)
        c = pltpu.async_copy(sc_hbm, sc_v, sem)
        d = pltpu.async_copy(sh_hbm, sh_v, sem)
        a.wait()
        b.wait()
        c.wait()
        d.wait()

        def body(i, carry):
            sl = pl.ds(i * L, L)
            idx = sp_v[sl]
            s = plsc.load_gather(sc_v, [idx])
            t = plsc.load_gather(sh_v, [idx])
            o_v[sl] = s * x_v[sl] + t
            return carry

        lax.fori_loop(0, iters, body, 0)

        pltpu.sync_copy(o_v, out_hbm.at[pl.ds(base, chunk)])

    return run(x, sc, sh, sp)


def kernel(in_field, scales, shifts, species_idx, ptr):
    del ptr
    n0 = in_field.shape[0]
    x = in_field.reshape(-1)
    sp = species_idx.reshape(-1).astype(jnp.int32)
    sc = scales.reshape(-1).astype(jnp.float32)
    sh = shifts.reshape(-1).astype(jnp.float32)

    info = plsc.get_sparse_core_info()
    nc, ns = info.num_cores, info.num_subcores
    nw = nc * ns

    n = n0
    if n % L != 0:
        n = (n0 + L - 1) // L * L
        x = jnp.pad(x, (0, n - n0))
        sp = jnp.pad(sp, (0, n - n0))

    per_worker = -(-n // nw)
    chunk = -(-per_worker // L) * L
    chunk = min(chunk, n)

    out = _scale_shift_call(x, sc, sh, sp, n, chunk, nc, ns)
    return out[:n0].reshape(n0, 1)


# speedup vs baseline: 3.8898x; 1.0033x over previous
import functools

import jax
import jax.numpy as jnp
from jax import lax
from jax.experimental import pallas as pl
from jax.experimental.pallas import tpu as pltpu
from jax.experimental.pallas import tpu_sc as plsc

L = 16


def _scale_shift_call(x, sc, sh, sp, n, chunk, nc, ns):
    half = chunk // 2
    iters = half // L

    mesh = plsc.VectorSubcoreMesh(core_axis_name="c", subcore_axis_name="s")

    @functools.partial(
        pl.kernel,
        mesh=mesh,
        out_type=jax.ShapeDtypeStruct((n,), jnp.float32),
        compiler_params=pltpu.CompilerParams(needs_layout_passes=False),
        scratch_types=[
            pltpu.VMEM((chunk,), jnp.int32),
            pltpu.VMEM((chunk,), jnp.float32),
            pltpu.VMEM((chunk,), jnp.float32),
            pltpu.VMEM((64,), jnp.float32),
            pltpu.VMEM((64,), jnp.float32),
            pltpu.SemaphoreType.DMA,
            pltpu.SemaphoreType.DMA,
            pltpu.SemaphoreType.DMA,
        ],
    )
    def run(
        x_hbm, sc_hbm, sh_hbm, sp_hbm, out_hbm,
        sp_v, x_v, o_v, sc_v, sh_v, sem0, sem1, osem,
    ):
        wid = lax.axis_index("s") * nc + lax.axis_index("c")
        base = jnp.minimum(wid * chunk, n - chunk)

        h0 = [
            pltpu.async_copy(sp_hbm.at[pl.ds(base, half)], sp_v.at[pl.ds(0, half)], sem0),
            pltpu.async_copy(x_hbm.at[pl.ds(base, half)], x_v.at[pl.ds(0, half)], sem0),
            pltpu.async_copy(sc_hbm, sc_v, sem0),
            pltpu.async_copy(sh_hbm, sh_v, sem0),
        ]
        h1 = [
            pltpu.async_copy(
                sp_hbm.at[pl.ds(base + half, half)], sp_v.at[pl.ds(half, half)], sem1
            ),
            pltpu.async_copy(
                x_hbm.at[pl.ds(base + half, half)], x_v.at[pl.ds(half, half)], sem1
            ),
        ]

        def body(start):
            def it(i, carry):
                sl = pl.ds(start + i * L, L)
                idx = sp_v[sl]
                s = plsc.load_gather(sc_v, [idx])
                t = plsc.load_gather(sh_v, [idx])
                o_v[sl] = s * x_v[sl] + t
                return carry

            lax.fori_loop(0, iters, it, 0)

        for cp in h0:
            cp.wait()
        body(0)
        out0 = pltpu.async_copy(
            o_v.at[pl.ds(0, half)], out_hbm.at[pl.ds(base, half)], osem
        )
        for cp in h1:
            cp.wait()
        body(half)
        out1 = pltpu.async_copy(
            o_v.at[pl.ds(half, half)], out_hbm.at[pl.ds(base + half, half)], osem
        )
        out0.wait()
        out1.wait()

    return run(x, sc, sh, sp)


def kernel(in_field, scales, shifts, species_idx, ptr):
    del ptr
    n0 = in_field.shape[0]
    x = in_field.reshape(-1)
    sp = species_idx.reshape(-1).astype(jnp.int32)
    sc = scales.reshape(-1).astype(jnp.float32)
    sh = shifts.reshape(-1).astype(jnp.float32)

    info = plsc.get_sparse_core_info()
    nc, ns = info.num_cores, info.num_subcores
    nw = nc * ns

    n = n0
    if n % L != 0:
        n = (n0 + L - 1) // L * L
        x = jnp.pad(x, (0, n - n0))
        sp = jnp.pad(sp, (0, n - n0))

    per_worker = -(-n // nw)
    chunk = -(-per_worker // (2 * L)) * (2 * L)
    chunk = min(chunk, n)

    out = _scale_shift_call(x, sc, sh, sp, n, chunk, nc, ns)
    return out[:n0].reshape(n0, 1)
